# baseline (speedup 1.0000x reference)
import functools
import math

import jax
import jax.numpy as jnp
from jax import lax
from jax.experimental import pallas as pl
from jax.experimental.pallas import tpu as pltpu
from jax.experimental.pallas import tpu_sc as plsc

N = 10000
D = 128
G = 16
NUM_TILES = 32
CH = 128
NPAD = 10112
RPT = NPAD // 16
RB = 1000
NB = N // RB

_mesh = plsc.VectorSubcoreMesh(core_axis_name="c", subcore_axis_name="s")


def _edges_per_tile(E_total):
  ept = -(-E_total // (NUM_TILES * 8 * CH)) * (8 * CH)
  return ept, ept * NUM_TILES


def _split_chunks(E_total, frac0):
  e0 = int(round(E_total * frac0))
  if frac0 >= 1.0:
    e0 = E_total
  npc0 = -(-e0 // (16 * SCH)) if e0 else 0
  npc0 = -(-npc0 // 8) * 8
  e1 = E_total - e0
  npc1 = -(-e1 // (16 * SCH)) if e1 else 0
  npc1 = -(-npc1 // 8) * 8
  return npc0, npc1, e0


def _make_deg_kernel(EPT):
  NCHUNK = EPT // CH

  @functools.partial(
      pl.kernel,
      mesh=_mesh,
      out_type=jax.ShapeDtypeStruct((2, NPAD, D), jnp.float32),
      scratch_types=[
          pltpu.VMEM((NCHUNK, CH), jnp.int32),
          pltpu.VMEM((CH, D), jnp.float32),
          pltpu.VMEM_SHARED((NPAD, D), jnp.float32),
          pltpu.SemaphoreType.DMA,
      ],
  )
  def deg_kernel(dst_hbm, ones_hbm, zeros_hbm, out_hbm, idst, ones_v, acc, sem):
    cid = lax.axis_index("c")
    sid = lax.axis_index("s")
    pltpu.sync_copy(zeros_hbm, acc.at[pl.ds(sid * RPT, RPT)])
    pltpu.sync_copy(ones_hbm, ones_v)
    cb = (cid * 16 + sid) * NCHUNK
    pltpu.sync_copy(dst_hbm.at[pl.ds(cb, NCHUNK)], idst)
    plsc.subcore_barrier()

    def body(t, carry):
      pltpu.async_copy(ones_v, acc.at[idst.at[t]], sem, add=True)
      return carry

    lax.fori_loop(0, NCHUNK, body, 0)

    def drain(t, carry):
      pltpu.make_async_copy(ones_v, acc.at[idst.at[t]], sem).wait()
      return carry

    lax.fori_loop(0, NCHUNK, drain, 0)
    plsc.subcore_barrier()
    pltpu.sync_copy(acc.at[pl.ds(sid * RPT, RPT)],
                    out_hbm.at[cid, pl.ds(sid * RPT, RPT)])

  return deg_kernel


SCH = 64
SLOTS = 4
PF = 2
NBUF = SLOTS


HCMAX = 40


def _phases(npc):
  if npc == 0:
    return 0, 0
  best = 8
  for k in range(8, HCMAX + 1, 8):
    if npc % k == 0:
      best = k
  return best, npc // best


def _make_scatter_kernel(NPC0, NPC1):
  @functools.partial(
      pl.kernel,
      mesh=_mesh,
      out_type=jax.ShapeDtypeStruct((2, NPAD, D), jnp.float32),
      scratch_types=[
          pltpu.VMEM((HCMAX, SCH), jnp.int32),
          pltpu.VMEM((HCMAX, SCH), jnp.int32),
          pltpu.VMEM_SHARED((NPAD, D), jnp.float32),
      ] + [pltpu.VMEM((SCH, D), jnp.float32)] * SLOTS
        + [pltpu.SemaphoreType.DMA] * (2 * SLOTS),
  )
  def scatter_kernel(rows_hbm, src_hbm, dst_hbm, zeros_hbm, out_hbm,
                     isrc, idst, acc, *rs):
    rows = rs[:SLOTS]
    gsem = rs[SLOTS:2 * SLOTS]
    ssem = rs[2 * SLOTS:]
    cid = lax.axis_index("c")
    sid = lax.axis_index("s")
    pltpu.sync_copy(zeros_hbm, acc.at[pl.ds(sid * RPT, RPT)])
    plsc.subcore_barrier()

    def go(npc, core_base):
      hc, nph = _phases(npc)
      tb = core_base + sid * npc
      for phase in range(nph):
        cb = tb + phase * hc
        pltpu.sync_copy(src_hbm.at[pl.ds(cb, hc)], isrc.at[pl.ds(0, hc)])
        pltpu.sync_copy(dst_hbm.at[pl.ds(cb, hc)], idst.at[pl.ds(0, hc)])
        for s in range(PF):
          pltpu.async_copy(rows_hbm.at[isrc.at[s]], rows[s], gsem[s])

        def body(tq, carry):
          for s in range(SLOTS):
            t = tq * SLOTS + s
            pltpu.make_async_copy(rows_hbm.at[isrc.at[t]], rows[s],
                                  gsem[s]).wait()
            pltpu.async_copy(rows[s], acc.at[idst.at[t]], ssem[s], add=True)
            s2 = (s + PF) % SLOTS

            @pl.when(t + PF < hc)
            def _pf():
              @pl.when(t >= PF)
              def _drain_old():
                pltpu.make_async_copy(rows[s2], acc.at[idst.at[0]],
                                      ssem[s2]).wait()

              pltpu.async_copy(rows_hbm.at[isrc.at[t + PF]], rows[s2],
                               gsem[s2])

          return carry

        lax.fori_loop(0, hc // SLOTS, body, 0)
        for s in range(SLOTS):
          pltpu.make_async_copy(rows[s], acc.at[idst.at[0]], ssem[s]).wait()
      return 0

    lax.cond(cid == 0, lambda: go(NPC0, 0), lambda: go(NPC1, 16 * NPC0))
    plsc.subcore_barrier()
    pltpu.sync_copy(acc.at[pl.ds(sid * RPT, RPT)],
                    out_hbm.at[cid, pl.ds(sid * RPT, RPT)])

  return scatter_kernel


def _dinv_block(degp):
  return lax.rsqrt(degp[0, :, 0:1] + degp[1, :, 0:1] + 1.0)


def _mm1_body(x_ref, w_ref, degp_ref, o_ref):
  d = _dinv_block(degp_ref[...])
  o_ref[...] = jnp.dot(x_ref[...], w_ref[...],
                       preferred_element_type=jnp.float32) * d


def _layer_body(aggp_ref, hws_ref, degp_ref, b_ref, w_ref, o_ref):
  d = _dinv_block(degp_ref[...])
  a = aggp_ref[0] + aggp_ref[1]
  h = jnp.maximum(d * (a + hws_ref[...]) + b_ref[...], 0.0)
  o_ref[...] = jnp.dot(h, w_ref[...], preferred_element_type=jnp.float32) * d


def _final_body(aggp_ref, hws_ref, degp_ref, b_ref, moh_ref,
                fw1_ref, fb1_ref, fw2_ref, fb2_ref, o_ref,
                sums_scr, cnt_scr):
  i = pl.program_id(0)

  @pl.when(i == 0)
  def _init():
    sums_scr[...] = jnp.zeros_like(sums_scr)
    cnt_scr[...] = jnp.zeros_like(cnt_scr)

  d = _dinv_block(degp_ref[...])
  a = aggp_ref[0] + aggp_ref[1]
  h = jnp.maximum(d * (a + hws_ref[...]) + b_ref[...], 0.0)
  m = moh_ref[...]
  dn = (((0,), (0,)), ((), ()))
  sums_scr[...] += lax.dot_general(m, h, dn,
                                   preferred_element_type=jnp.float32)
  cnt_scr[...] += lax.dot_general(m, jnp.ones((RB, 8), jnp.float32), dn,
                                  preferred_element_type=jnp.float32)

  @pl.when(i == NB - 1)
  def _fin():
    pooled = sums_scr[...] / jnp.maximum(cnt_scr[...][:, 0:1], 1.0)
    y = jnp.maximum(
        jnp.dot(pooled, fw1_ref[...], preferred_element_type=jnp.float32)
        + fb1_ref[...], 0.0)
    o_ref[...] = jnp.dot(y, fw2_ref[...],
                         preferred_element_type=jnp.float32) + fb2_ref[...]


_rowspec = pl.BlockSpec((RB, D), lambda i: (i, 0))
_aggspec = pl.BlockSpec((2, RB, D), lambda i: (0, i, 0))
_degspec = pl.BlockSpec((2, RB, D), lambda i: (0, i, 0))
_wspec = pl.BlockSpec((D, D), lambda i: (0, 0))
_bspec = pl.BlockSpec((1, D), lambda i: (0, 0))

_mm1_call = pl.pallas_call(
    _mm1_body,
    grid=(NB,),
    in_specs=[_rowspec, _wspec, _degspec],
    out_specs=_rowspec,
    out_shape=jax.ShapeDtypeStruct((N, D), jnp.float32),
)

_layer_call = pl.pallas_call(
    _layer_body,
    grid=(NB,),
    in_specs=[_aggspec, _rowspec, _degspec, _bspec, _wspec],
    out_specs=_rowspec,
    out_shape=jax.ShapeDtypeStruct((N, D), jnp.float32),
)

_final_call = pl.pallas_call(
    _final_body,
    grid=(NB,),
    in_specs=[
        _aggspec, _rowspec, _degspec, _bspec,
        pl.BlockSpec((RB, G), lambda i: (i, 0)),
        pl.BlockSpec((D, D), lambda i: (0, 0)),
        _bspec,
        pl.BlockSpec((D, 8), lambda i: (0, 0)),
        pl.BlockSpec((1, 8), lambda i: (0, 0)),
    ],
    out_specs=pl.BlockSpec((G, 8), lambda i: (0, 0)),
    out_shape=jax.ShapeDtypeStruct((G, 8), jnp.float32),
    scratch_shapes=[
        pltpu.VMEM((G, D), jnp.float32),
        pltpu.VMEM((G, 8), jnp.float32),
    ],
)


F0 = 0.5


def _pad_region(a, fill, target):
  return jnp.concatenate([a, jnp.full((target - a.shape[0],), fill, a.dtype)])


def kernel(x, edge_index, batch, W1, b1, W2, b2, W3, b3, fW1, fb1, fW2, fb2):
  E = edge_index.shape[1]
  EPT, EPAD = _edges_per_tile(E)
  npadextra = EPAD - E
  dst_deg = jnp.concatenate(
      [edge_index[1], jnp.full((npadextra,), N, jnp.int32)]).reshape(-1, CH)

  NPC0, NPC1, e0 = _split_chunks(E, F0)
  c0, c1 = 16 * NPC0 * SCH, 16 * NPC1 * SCH
  src_flat = jnp.concatenate(
      [_pad_region(edge_index[0, :e0], 0, c0),
       _pad_region(edge_index[0, e0:], 0, c1)])
  dst_flat = jnp.concatenate(
      [_pad_region(edge_index[1, :e0], N, c0),
       _pad_region(edge_index[1, e0:], N, c1)])
  src_p = src_flat.reshape(-1, SCH)
  dst_p = dst_flat.reshape(-1, SCH)

  onesD = jnp.ones((CH, D), jnp.float32)
  zerosD = jnp.zeros((RPT, D), jnp.float32)
  moh = (batch[:, None] == jnp.arange(G, dtype=batch.dtype)[None, :])
  moh = moh.astype(jnp.float32)

  deg_call = _make_deg_kernel(EPT)
  sc_call = _make_scatter_kernel(NPC0, NPC1)

  degp = deg_call(dst_deg, onesD, zerosD)

  b1r = b1.reshape(1, D)
  b2r = b2.reshape(1, D)
  b3r = b3.reshape(1, D)

  hws1 = _mm1_call(x, W1, degp)
  agg1 = sc_call(hws1, src_p, dst_p, zerosD)
  hws2 = _layer_call(agg1, hws1, degp, b1r, W2)
  agg2 = sc_call(hws2, src_p, dst_p, zerosD)
  hws3 = _layer_call(agg2, hws2, degp, b2r, W3)
  agg3 = sc_call(hws3, src_p, dst_p, zerosD)
  out = _final_call(agg3, hws3, degp, b3r, moh,
                    fW1, fb1.reshape(1, D), fW2, fb2.reshape(1, 8))
  return out


# speedup vs baseline: 1.0399x; 1.0399x over previous
import functools
import math

import jax
import jax.numpy as jnp
from jax import lax
from jax.experimental import pallas as pl
from jax.experimental.pallas import tpu as pltpu
from jax.experimental.pallas import tpu_sc as plsc

N = 10000
D = 128
G = 16
NUM_TILES = 32
CH = 128
NPAD = 10240
RPT = NPAD // 16
RB = 2000
NB = N // RB

_mesh = plsc.VectorSubcoreMesh(core_axis_name="c", subcore_axis_name="s")


def _edges_per_tile(E_total):
  ept = -(-E_total // (NUM_TILES * 8 * CH)) * (8 * CH)
  return ept, ept * NUM_TILES


def _split_chunks(E_total, frac0):
  e0 = int(round(E_total * frac0))
  if frac0 >= 1.0:
    e0 = E_total
  npc0 = -(-e0 // (16 * SCH)) if e0 else 0
  npc0 = -(-npc0 // 8) * 8
  e1 = E_total - e0
  npc1 = -(-e1 // (16 * SCH)) if e1 else 0
  npc1 = -(-npc1 // 8) * 8
  return npc0, npc1, e0


def _make_deg_kernel(EPT):
  NCHUNK = EPT // CH

  @functools.partial(
      pl.kernel,
      mesh=_mesh,
      out_type=jax.ShapeDtypeStruct((2, NPAD, D), jnp.float32),
      scratch_types=[
          pltpu.VMEM((NCHUNK, CH), jnp.int32),
          pltpu.VMEM((CH, D), jnp.float32),
          pltpu.VMEM_SHARED((NPAD, D), jnp.float32),
          pltpu.SemaphoreType.DMA,
      ],
  )
  def deg_kernel(dst_hbm, ones_hbm, zeros_hbm, out_hbm, idst, ones_v, acc, sem):
    cid = lax.axis_index("c")
    sid = lax.axis_index("s")
    pltpu.sync_copy(zeros_hbm, acc.at[pl.ds(sid * RPT, RPT)])
    pltpu.sync_copy(ones_hbm, ones_v)
    cb = (cid * 16 + sid) * NCHUNK
    pltpu.sync_copy(dst_hbm.at[pl.ds(cb, NCHUNK)], idst)
    plsc.subcore_barrier()

    def body(t, carry):
      pltpu.async_copy(ones_v, acc.at[idst.at[t]], sem, add=True)
      return carry

    lax.fori_loop(0, NCHUNK, body, 0)

    def drain(t, carry):
      pltpu.make_async_copy(ones_v, acc.at[idst.at[t]], sem).wait()
      return carry

    lax.fori_loop(0, NCHUNK, drain, 0)
    plsc.subcore_barrier()
    pltpu.sync_copy(acc.at[pl.ds(sid * RPT, RPT)],
                    out_hbm.at[cid, pl.ds(sid * RPT, RPT)])

  return deg_kernel


SCH = 64
SLOTS = 4
PF = 2
NBUF = SLOTS


HCMAX = 40


def _phases(npc):
  if npc == 0:
    return 0, 0
  best = 8
  for k in range(8, HCMAX + 1, 8):
    if npc % k == 0:
      best = k
  return best, npc // best


def _make_scatter_kernel(NPC0, NPC1):
  @functools.partial(
      pl.kernel,
      mesh=_mesh,
      out_type=jax.ShapeDtypeStruct((2, NPAD, D), jnp.float32),
      scratch_types=[
          pltpu.VMEM((HCMAX, SCH), jnp.int32),
          pltpu.VMEM((HCMAX, SCH), jnp.int32),
          pltpu.VMEM_SHARED((NPAD, D), jnp.float32),
      ] + [pltpu.VMEM((SCH, D), jnp.float32)] * SLOTS
        + [pltpu.SemaphoreType.DMA] * SLOTS,
  )
  def scatter_kernel(rows_hbm, src_hbm, dst_hbm, zeros_hbm, out_hbm,
                     isrc, idst, acc, *rs):
    rows = rs[:SLOTS]
    gsem = rs[SLOTS:]
    cid = lax.axis_index("c")
    sid = lax.axis_index("s")
    pltpu.sync_copy(zeros_hbm, acc.at[pl.ds(sid * RPT, RPT)])
    plsc.subcore_barrier()

    def go(npc, core_base):
      hc, nph = _phases(npc)
      tb = core_base + sid * npc
      for phase in range(nph):
        cb = tb + phase * hc
        pltpu.sync_copy(src_hbm.at[pl.ds(cb, hc)], isrc.at[pl.ds(0, hc)])
        pltpu.sync_copy(dst_hbm.at[pl.ds(cb, hc)], idst.at[pl.ds(0, hc)])
        for s in range(SLOTS):
          pltpu.async_copy(rows_hbm.at[isrc.at[s]], rows[s], gsem[s])

        def body(tq, carry):
          for s in range(SLOTS):
            t = tq * SLOTS + s
            pltpu.make_async_copy(rows_hbm.at[isrc.at[t]], rows[s],
                                  gsem[s]).wait()
            pltpu.sync_copy(rows[s], acc.at[idst.at[t]], add=True)

            @pl.when(t + SLOTS < hc)
            def _pf():
              pltpu.async_copy(rows_hbm.at[isrc.at[t + SLOTS]], rows[s],
                               gsem[s])

          return carry

        lax.fori_loop(0, hc // SLOTS, body, 0)
      return 0

    lax.cond(cid == 0, lambda: go(NPC0, 0), lambda: go(NPC1, 16 * NPC0))
    plsc.subcore_barrier()
    pltpu.sync_copy(acc.at[pl.ds(sid * RPT, RPT)],
                    out_hbm.at[cid, pl.ds(sid * RPT, RPT)])

  return scatter_kernel


def _dinv_block(degp):
  return lax.rsqrt(degp[0, :, 0:1] + degp[1, :, 0:1] + 1.0)


def _mm1_body(x_ref, w_ref, degp_ref, o_ref):
  d = _dinv_block(degp_ref[...])
  o_ref[...] = jnp.dot(x_ref[...], w_ref[...],
                       preferred_element_type=jnp.float32) * d


def _layer_body(aggp_ref, hws_ref, degp_ref, b_ref, w_ref, o_ref):
  d = _dinv_block(degp_ref[...])
  a = aggp_ref[0].astype(jnp.float32) + aggp_ref[1].astype(jnp.float32)
  h = jnp.maximum(d * (a + hws_ref[...]) + b_ref[...], 0.0)
  o_ref[...] = jnp.dot(h, w_ref[...], preferred_element_type=jnp.float32) * d


def _final_body(aggp_ref, hws_ref, degp_ref, b_ref, moh_ref,
                fw1_ref, fb1_ref, fw2_ref, fb2_ref, o_ref,
                sums_scr, cnt_scr):
  i = pl.program_id(0)

  @pl.when(i == 0)
  def _init():
    sums_scr[...] = jnp.zeros_like(sums_scr)
    cnt_scr[...] = jnp.zeros_like(cnt_scr)

  d = _dinv_block(degp_ref[...])
  a = aggp_ref[0].astype(jnp.float32) + aggp_ref[1].astype(jnp.float32)
  h = jnp.maximum(d * (a + hws_ref[...]) + b_ref[...], 0.0)
  m = moh_ref[...]
  dn = (((0,), (0,)), ((), ()))
  sums_scr[...] += lax.dot_general(m, h, dn,
                                   preferred_element_type=jnp.float32)
  cnt_scr[...] += lax.dot_general(m, jnp.ones((RB, 8), jnp.float32), dn,
                                  preferred_element_type=jnp.float32)

  @pl.when(i == NB - 1)
  def _fin():
    pooled = sums_scr[...] / jnp.maximum(cnt_scr[...][:, 0:1], 1.0)
    y = jnp.maximum(
        jnp.dot(pooled, fw1_ref[...], preferred_element_type=jnp.float32)
        + fb1_ref[...], 0.0)
    o_ref[...] = jnp.dot(y, fw2_ref[...],
                         preferred_element_type=jnp.float32) + fb2_ref[...]


_rowspec = pl.BlockSpec((RB, D), lambda i: (i, 0))
_aggspec = pl.BlockSpec((2, RB, D), lambda i: (0, i, 0))
_degspec = pl.BlockSpec((2, RB, D), lambda i: (0, i, 0))
_wspec = pl.BlockSpec((D, D), lambda i: (0, 0))
_bspec = pl.BlockSpec((1, D), lambda i: (0, 0))

_mm1_call = pl.pallas_call(
    _mm1_body,
    grid=(NB,),
    in_specs=[_rowspec, _wspec, _degspec],
    out_specs=_rowspec,
    out_shape=jax.ShapeDtypeStruct((N, D), jnp.float32),
)

_layer_call = pl.pallas_call(
    _layer_body,
    grid=(NB,),
    in_specs=[_aggspec, _rowspec, _degspec, _bspec, _wspec],
    out_specs=_rowspec,
    out_shape=jax.ShapeDtypeStruct((N, D), jnp.float32),
)

_final_call = pl.pallas_call(
    _final_body,
    grid=(NB,),
    in_specs=[
        _aggspec, _rowspec, _degspec, _bspec,
        pl.BlockSpec((RB, G), lambda i: (i, 0)),
        pl.BlockSpec((D, D), lambda i: (0, 0)),
        _bspec,
        pl.BlockSpec((D, 8), lambda i: (0, 0)),
        pl.BlockSpec((1, 8), lambda i: (0, 0)),
    ],
    out_specs=pl.BlockSpec((G, 8), lambda i: (0, 0)),
    out_shape=jax.ShapeDtypeStruct((G, 8), jnp.float32),
    scratch_shapes=[
        pltpu.VMEM((G, D), jnp.float32),
        pltpu.VMEM((G, 8), jnp.float32),
    ],
)


F0 = 0.5


def _pad_region(a, fill, target):
  return jnp.concatenate([a, jnp.full((target - a.shape[0],), fill, a.dtype)])


def kernel(x, edge_index, batch, W1, b1, W2, b2, W3, b3, fW1, fb1, fW2, fb2):
  E = edge_index.shape[1]
  EPT, EPAD = _edges_per_tile(E)
  npadextra = EPAD - E
  dst_deg = jnp.concatenate(
      [edge_index[1], jnp.full((npadextra,), N, jnp.int32)]).reshape(-1, CH)

  NPC0, NPC1, e0 = _split_chunks(E, F0)
  c0, c1 = 16 * NPC0 * SCH, 16 * NPC1 * SCH
  src_flat = jnp.concatenate(
      [_pad_region(edge_index[0, :e0], 0, c0),
       _pad_region(edge_index[0, e0:], 0, c1)])
  dst_flat = jnp.concatenate(
      [_pad_region(edge_index[1, :e0], N, c0),
       _pad_region(edge_index[1, e0:], N, c1)])
  src_p = src_flat.reshape(-1, SCH)
  dst_p = dst_flat.reshape(-1, SCH)

  onesD = jnp.ones((CH, D), jnp.float32)
  zerosD = jnp.zeros((RPT, D), jnp.float32)
  moh = (batch[:, None] == jnp.arange(G, dtype=batch.dtype)[None, :])
  moh = moh.astype(jnp.float32)

  deg_call = _make_deg_kernel(EPT)
  sc_call = _make_scatter_kernel(NPC0, NPC1)

  degp = deg_call(dst_deg, onesD, zerosD)

  b1r = b1.reshape(1, D)
  b2r = b2.reshape(1, D)
  b3r = b3.reshape(1, D)

  hws1 = _mm1_call(x, W1, degp)
  agg1 = sc_call(hws1, src_p, dst_p, zerosD)
  hws2 = _layer_call(agg1, hws1, degp, b1r, W2)
  agg2 = sc_call(hws2, src_p, dst_p, zerosD)
  hws3 = _layer_call(agg2, hws2, degp, b2r, W3)
  agg3 = sc_call(hws3, src_p, dst_p, zerosD)
  out = _final_call(agg3, hws3, degp, b3r, moh,
                    fW1, fb1.reshape(1, D), fW2, fb2.reshape(1, 8))
  return out
